# baseline (speedup 1.0000x reference)
import jax
import jax.numpy as jnp
from jax import lax
from jax.experimental import pallas as pl
from jax.experimental.pallas import tpu as pltpu
from jax.experimental.pallas import tpu_sc as plsc

N = 10000
E = 320000
DIN = 3703
DH = 128
DO = 64

NC = 2
NS = 16
NW = NC * NS
CH = 128
K = -(-E // (NW * CH))
if K % 2:
    K += 1
CAP = NW * CH * K
NPAD = N + 112
SLICE = NPAD // NS

ROWB = 400
NB = N // ROWB

_mesh = plsc.VectorSubcoreMesh(core_axis_name="c", subcore_axis_name="s")
_sc_params = pltpu.CompilerParams(use_tc_tiling_on_sc=False)


def _deg_body(dst_hbm, ones_hbm, zeros_hbm, out_hbm, idx_v, ones_v, deg_sh):
    c = lax.axis_index("c")
    s = lax.axis_index("s")
    w = s * NC + c
    pltpu.sync_copy(zeros_hbm.at[pl.ds(s * SLICE, SLICE)],
                    deg_sh.at[pl.ds(s * SLICE, SLICE)])
    pltpu.sync_copy(dst_hbm.at[w], idx_v)
    pltpu.sync_copy(ones_hbm, ones_v)
    plsc.subcore_barrier()

    @pl.loop(0, K)
    def _(j):
        pltpu.sync_copy(ones_v, deg_sh.at[idx_v.at[j]], add=True)

    plsc.subcore_barrier()
    pltpu.sync_copy(deg_sh.at[pl.ds(s * SLICE, SLICE)],
                    out_hbm.at[c, pl.ds(s * SLICE, SLICE)])


_deg_kernel = pl.kernel(
    _deg_body,
    out_type=jax.ShapeDtypeStruct((NC, NPAD, 16), jnp.float32),
    mesh=_mesh,
    compiler_params=_sc_params,
    scratch_types=[
        pltpu.VMEM((K, CH), jnp.int32),
        pltpu.VMEM((CH, 16), jnp.float32),
        pltpu.VMEM_SHARED((NPAD, 16), jnp.float32),
    ],
)


def _make_scatter(d):
    NBUF = 8
    DELTA = NBUF // 2

    def body(src_hbm, dst_hbm, g_hbm, zeros_hbm, out_hbm,
             srcv, dstv, *rest):
        bufs = rest[:NBUF]
        gsems = rest[NBUF:2 * NBUF]
        ssems = rest[2 * NBUF:3 * NBUF]
        acc_sh = rest[3 * NBUF]
        c = lax.axis_index("c")
        s = lax.axis_index("s")
        w = s * NC + c
        pltpu.sync_copy(zeros_hbm.at[pl.ds(s * SLICE, SLICE)],
                        acc_sh.at[pl.ds(s * SLICE, SLICE)])
        pltpu.sync_copy(src_hbm.at[w], srcv)
        pltpu.sync_copy(dst_hbm.at[w], dstv)
        plsc.subcore_barrier()

        for b in range(DELTA):
            pltpu.async_copy(g_hbm.at[srcv.at[b]], bufs[b], gsems[b])

        @pl.loop(0, K // NBUF)
        def _(jj):
            j0 = NBUF * jj
            for b in range(NBUF):
                j = j0 + b
                f = (b + DELTA) % NBUF

                @pl.when(j + DELTA >= NBUF)
                def _():
                    pltpu.make_async_copy(
                        bufs[f], acc_sh.at[dstv.at[j + DELTA - NBUF]],
                        ssems[f]).wait()

                @pl.when(j + DELTA < K)
                def _():
                    pltpu.async_copy(
                        g_hbm.at[srcv.at[j + DELTA]], bufs[f], gsems[f])

                pltpu.make_async_copy(
                    g_hbm.at[srcv.at[j]], bufs[b], gsems[b]).wait()
                pltpu.async_copy(bufs[b], acc_sh.at[dstv.at[j]], ssems[b],
                                 add=True)

        for b in range(DELTA, NBUF):
            pltpu.make_async_copy(
                bufs[b], acc_sh.at[dstv.at[K - NBUF + b]], ssems[b]).wait()

        plsc.subcore_barrier()
        pltpu.sync_copy(acc_sh.at[pl.ds(s * SLICE, SLICE)],
                        out_hbm.at[c, pl.ds(s * SLICE, SLICE)])

    return pl.kernel(
        body,
        out_type=jax.ShapeDtypeStruct((NC, NPAD, d), jnp.float32),
        mesh=_mesh,
        compiler_params=_sc_params,
        scratch_types=[
            pltpu.VMEM((K, CH), jnp.int32),
            pltpu.VMEM((K, CH), jnp.int32),
        ] + [pltpu.VMEM((CH, d), jnp.float32) for _ in range(NBUF)]
          + [pltpu.SemaphoreType.DMA for _ in range(2 * NBUF)]
          + [pltpu.VMEM_SHARED((NPAD, d), jnp.float32)],
    )


_scatter64 = _make_scatter(DO)
_scatter64_b = _make_scatter(DO)


def _mm1_body(x_ref, w_ref, degp_ref, ga_ref, gb_ref, dis_ref):
    h = jnp.dot(x_ref[...], w_ref[...], preferred_element_type=jnp.float32)
    degp = degp_ref[...]
    deg = degp[0, :, 0] + degp[1, :, 0] + 1.0
    dis = lax.rsqrt(deg)[:, None]
    g = h * dis
    ga_ref[...] = g[:, :DO]
    gb_ref[...] = g[:, DO:]
    dis_ref[...] = dis


_mm1 = pl.pallas_call(
    _mm1_body,
    grid=(NB,),
    in_specs=[
        pl.BlockSpec((ROWB, DIN), lambda i: (i, 0)),
        pl.BlockSpec((DIN, DH), lambda i: (0, 0)),
        pl.BlockSpec((NC, ROWB, 16), lambda i: (0, i, 0)),
    ],
    out_specs=[
        pl.BlockSpec((ROWB, DO), lambda i: (i, 0)),
        pl.BlockSpec((ROWB, DO), lambda i: (i, 0)),
        pl.BlockSpec((ROWB, 1), lambda i: (i, 0)),
    ],
    out_shape=[
        jax.ShapeDtypeStruct((N, DO), jnp.float32),
        jax.ShapeDtypeStruct((N, DO), jnp.float32),
        jax.ShapeDtypeStruct((N, 1), jnp.float32),
    ],
)


def _ep1_body(acca_ref, accb_ref, ga_ref, gb_ref, dis_ref, b1_ref, w2_ref,
              g2_ref):
    acca = acca_ref[...]
    accb = accb_ref[...]
    dis = dis_ref[...]
    t = jnp.concatenate(
        [acca[0] + acca[1] + ga_ref[...], accb[0] + accb[1] + gb_ref[...]],
        axis=1) * dis + b1_ref[...]
    t = jnp.maximum(t, 0.0)
    h2 = jnp.dot(t, w2_ref[...], preferred_element_type=jnp.float32)
    g2_ref[...] = h2 * dis


_ep1 = pl.pallas_call(
    _ep1_body,
    grid=(NB,),
    in_specs=[
        pl.BlockSpec((NC, ROWB, DO), lambda i: (0, i, 0)),
        pl.BlockSpec((NC, ROWB, DO), lambda i: (0, i, 0)),
        pl.BlockSpec((ROWB, DO), lambda i: (i, 0)),
        pl.BlockSpec((ROWB, DO), lambda i: (i, 0)),
        pl.BlockSpec((ROWB, 1), lambda i: (i, 0)),
        pl.BlockSpec((1, DH), lambda i: (0, 0)),
        pl.BlockSpec((DH, DO), lambda i: (0, 0)),
    ],
    out_specs=pl.BlockSpec((ROWB, DO), lambda i: (i, 0)),
    out_shape=jax.ShapeDtypeStruct((N, DO), jnp.float32),
)


def _ep2_body(accp_ref, g2_ref, dis_ref, b2_ref, z_ref):
    accp = accp_ref[...]
    z_ref[...] = (accp[0] + accp[1] + g2_ref[...]) * dis_ref[...] + b2_ref[...]


_ep2 = pl.pallas_call(
    _ep2_body,
    grid=(NB,),
    in_specs=[
        pl.BlockSpec((NC, ROWB, DO), lambda i: (0, i, 0)),
        pl.BlockSpec((ROWB, DO), lambda i: (i, 0)),
        pl.BlockSpec((ROWB, 1), lambda i: (i, 0)),
        pl.BlockSpec((1, DO), lambda i: (0, 0)),
    ],
    out_specs=pl.BlockSpec((ROWB, DO), lambda i: (i, 0)),
    out_shape=jax.ShapeDtypeStruct((N, DO), jnp.float32),
)


def kernel(x, edge_index, W1, b1, W2, b2):
    src = edge_index[0]
    dst = edge_index[1]
    npad = CAP - E
    pad = jnp.arange(npad, dtype=jnp.int32)
    src_t = jnp.concatenate([src, pad % N]).reshape(NW, K, CH)
    dst_t = jnp.concatenate([dst, N + pad % (NPAD - N)]).reshape(NW, K, CH)
    zeros16 = jnp.zeros((NPAD, 16), jnp.float32)
    zeros_o = jnp.zeros((NPAD, DO), jnp.float32)
    ones16 = jnp.ones((CH, 16), jnp.float32)

    deg_parts = _deg_kernel(dst_t, ones16, zeros16)
    g1a, g1b, dis = _mm1(x, W1, deg_parts)
    acc1a = _scatter64(src_t, dst_t, g1a, zeros_o)
    acc1b = _scatter64_b(src_t, dst_t, g1b, zeros_o)
    g2 = _ep1(acc1a, acc1b, g1a, g1b, dis,
              b1.reshape(1, DH), W2)
    acc2 = _scatter64(src_t, dst_t, g2, zeros_o)
    z = _ep2(acc2, g2, dis, b2.reshape(1, DO))
    return z


# speedup vs baseline: 1.0261x; 1.0261x over previous
import jax
import jax.numpy as jnp
from jax import lax
from jax.experimental import pallas as pl
from jax.experimental.pallas import tpu as pltpu
from jax.experimental.pallas import tpu_sc as plsc

N = 10000
E = 320000
DIN = 3703
DH = 128
DO = 64

NC = 2
NS = 16
NW = NC * NS
CH = 128
K = -(-E // (NW * CH))
if K % 2:
    K += 1
CAP = NW * CH * K
NPAD = N + 112
SLICE = NPAD // NS

ROWB = 400
NB = N // ROWB

_mesh = plsc.VectorSubcoreMesh(core_axis_name="c", subcore_axis_name="s")
_sc_params = pltpu.CompilerParams(use_tc_tiling_on_sc=False)


def _deg_body(dst_hbm, ones_hbm, zeros_hbm, out_hbm, idx_v, ones_v, deg_sh):
    c = lax.axis_index("c")
    s = lax.axis_index("s")
    w = s * NC + c
    pltpu.sync_copy(zeros_hbm.at[pl.ds(s * SLICE, SLICE)],
                    deg_sh.at[pl.ds(s * SLICE, SLICE)])
    pltpu.sync_copy(dst_hbm.at[w], idx_v)
    pltpu.sync_copy(ones_hbm, ones_v)
    plsc.subcore_barrier()

    @pl.loop(0, K)
    def _(j):
        pltpu.sync_copy(ones_v, deg_sh.at[idx_v.at[j]], add=True)

    plsc.subcore_barrier()
    pltpu.sync_copy(deg_sh.at[pl.ds(s * SLICE, SLICE)],
                    out_hbm.at[c, pl.ds(s * SLICE, SLICE)])


_deg_kernel = pl.kernel(
    _deg_body,
    out_type=jax.ShapeDtypeStruct((NC, NPAD, 16), jnp.float32),
    mesh=_mesh,
    compiler_params=_sc_params,
    scratch_types=[
        pltpu.VMEM((K, CH), jnp.int32),
        pltpu.VMEM((CH, 16), jnp.float32),
        pltpu.VMEM_SHARED((NPAD, 16), jnp.float32),
    ],
)


def _make_scatter(d):
    NBUF = 4

    def body(src_hbm, dst_hbm, g_hbm, zeros_hbm, out_hbm,
             srcv, dstv, *rest):
        bufs = rest[:NBUF]
        sems = rest[NBUF:2 * NBUF]
        acc_sh = rest[2 * NBUF]
        c = lax.axis_index("c")
        s = lax.axis_index("s")
        w = s * NC + c
        pltpu.sync_copy(zeros_hbm.at[pl.ds(s * SLICE, SLICE)],
                        acc_sh.at[pl.ds(s * SLICE, SLICE)])
        pltpu.sync_copy(src_hbm.at[w], srcv)
        pltpu.sync_copy(dst_hbm.at[w], dstv)
        plsc.subcore_barrier()

        for b in range(NBUF):
            pltpu.make_async_copy(g_hbm.at[srcv.at[b]], bufs[b], sems[b]).start()

        @pl.loop(0, K // NBUF)
        def _(jj):
            j = NBUF * jj
            for b in range(NBUF):
                pltpu.make_async_copy(
                    g_hbm.at[srcv.at[j + b]], bufs[b], sems[b]).wait()
                pltpu.sync_copy(bufs[b], acc_sh.at[dstv.at[j + b]], add=True)

                @pl.when(j + b + NBUF < K)
                def _():
                    pltpu.make_async_copy(
                        g_hbm.at[srcv.at[j + b + NBUF]], bufs[b],
                        sems[b]).start()

        plsc.subcore_barrier()
        pltpu.sync_copy(acc_sh.at[pl.ds(s * SLICE, SLICE)],
                        out_hbm.at[c, pl.ds(s * SLICE, SLICE)])

    return pl.kernel(
        body,
        out_type=jax.ShapeDtypeStruct((NC, NPAD, d), jnp.float32),
        mesh=_mesh,
        compiler_params=_sc_params,
        scratch_types=[
            pltpu.VMEM((K, CH), jnp.int32),
            pltpu.VMEM((K, CH), jnp.int32),
        ] + [pltpu.VMEM((CH, d), jnp.float32) for _ in range(NBUF)]
          + [pltpu.SemaphoreType.DMA for _ in range(NBUF)]
          + [pltpu.VMEM_SHARED((NPAD, d), jnp.float32)],
    )


_scatter64 = _make_scatter(DO)
_scatter64_b = _make_scatter(DO)


def _mm1_body(x_ref, w_ref, degp_ref, ga_ref, gb_ref, dis_ref):
    h = jnp.dot(x_ref[...], w_ref[...], preferred_element_type=jnp.float32)
    degp = degp_ref[...]
    deg = degp[0, :, 0] + degp[1, :, 0] + 1.0
    dis = lax.rsqrt(deg)[:, None]
    g = h * dis
    ga_ref[...] = g[:, :DO]
    gb_ref[...] = g[:, DO:]
    dis_ref[...] = dis


_mm1 = pl.pallas_call(
    _mm1_body,
    grid=(NB,),
    in_specs=[
        pl.BlockSpec((ROWB, DIN), lambda i: (i, 0)),
        pl.BlockSpec((DIN, DH), lambda i: (0, 0)),
        pl.BlockSpec((NC, ROWB, 16), lambda i: (0, i, 0)),
    ],
    out_specs=[
        pl.BlockSpec((ROWB, DO), lambda i: (i, 0)),
        pl.BlockSpec((ROWB, DO), lambda i: (i, 0)),
        pl.BlockSpec((ROWB, 1), lambda i: (i, 0)),
    ],
    out_shape=[
        jax.ShapeDtypeStruct((N, DO), jnp.float32),
        jax.ShapeDtypeStruct((N, DO), jnp.float32),
        jax.ShapeDtypeStruct((N, 1), jnp.float32),
    ],
)


def _ep1_body(acca_ref, accb_ref, ga_ref, gb_ref, dis_ref, b1_ref, w2_ref,
              g2_ref):
    acca = acca_ref[...]
    accb = accb_ref[...]
    dis = dis_ref[...]
    t = jnp.concatenate(
        [acca[0] + acca[1] + ga_ref[...], accb[0] + accb[1] + gb_ref[...]],
        axis=1) * dis + b1_ref[...]
    t = jnp.maximum(t, 0.0)
    h2 = jnp.dot(t, w2_ref[...], preferred_element_type=jnp.float32)
    g2_ref[...] = h2 * dis


_ep1 = pl.pallas_call(
    _ep1_body,
    grid=(NB,),
    in_specs=[
        pl.BlockSpec((NC, ROWB, DO), lambda i: (0, i, 0)),
        pl.BlockSpec((NC, ROWB, DO), lambda i: (0, i, 0)),
        pl.BlockSpec((ROWB, DO), lambda i: (i, 0)),
        pl.BlockSpec((ROWB, DO), lambda i: (i, 0)),
        pl.BlockSpec((ROWB, 1), lambda i: (i, 0)),
        pl.BlockSpec((1, DH), lambda i: (0, 0)),
        pl.BlockSpec((DH, DO), lambda i: (0, 0)),
    ],
    out_specs=pl.BlockSpec((ROWB, DO), lambda i: (i, 0)),
    out_shape=jax.ShapeDtypeStruct((N, DO), jnp.float32),
)


def _ep2_body(accp_ref, g2_ref, dis_ref, b2_ref, z_ref):
    accp = accp_ref[...]
    z_ref[...] = (accp[0] + accp[1] + g2_ref[...]) * dis_ref[...] + b2_ref[...]


_ep2 = pl.pallas_call(
    _ep2_body,
    grid=(NB,),
    in_specs=[
        pl.BlockSpec((NC, ROWB, DO), lambda i: (0, i, 0)),
        pl.BlockSpec((ROWB, DO), lambda i: (i, 0)),
        pl.BlockSpec((ROWB, 1), lambda i: (i, 0)),
        pl.BlockSpec((1, DO), lambda i: (0, 0)),
    ],
    out_specs=pl.BlockSpec((ROWB, DO), lambda i: (i, 0)),
    out_shape=jax.ShapeDtypeStruct((N, DO), jnp.float32),
)


def kernel(x, edge_index, W1, b1, W2, b2):
    src = edge_index[0]
    dst = edge_index[1]
    npad = CAP - E
    pad = jnp.arange(npad, dtype=jnp.int32)
    src_t = jnp.concatenate([src, pad % N]).reshape(NW, K, CH)
    dst_t = jnp.concatenate([dst, N + pad % (NPAD - N)]).reshape(NW, K, CH)
    zeros16 = jnp.zeros((NPAD, 16), jnp.float32)
    zeros_o = jnp.zeros((NPAD, DO), jnp.float32)
    ones16 = jnp.ones((CH, 16), jnp.float32)

    deg_parts = _deg_kernel(dst_t, ones16, zeros16)
    g1a, g1b, dis = _mm1(x, W1, deg_parts)
    acc1a = _scatter64(src_t, dst_t, g1a, zeros_o)
    acc1b = _scatter64_b(src_t, dst_t, g1b, zeros_o)
    g2 = _ep1(acc1a, acc1b, g1a, g1b, dis,
              b1.reshape(1, DH), W2)
    acc2 = _scatter64(src_t, dst_t, g2, zeros_o)
    z = _ep2(acc2, g2, dis, b2.reshape(1, DO))
    return z
